# baseline (device time: 17982 ns/iter reference)
import jax
import jax.numpy as jnp
from jax import lax
from jax.experimental import pallas as pl
from jax.experimental.pallas import tpu as pltpu

B, H, D, BS = 8, 8, 64, 16
SLOTS = 64
PAGES_LOCAL = 64
T = PAGES_LOCAL * BS
HALF = H // 2
NEG = -1e30


def kernel(Q, K, V, bt, lens):
    def body(q_hbm, k_hbm, v_hbm, bt_hbm, lens_hbm, out_hbm,
             q_vm, k_vm, v_vm, bt_vm, lens_sm, out_vm,
             o_buf, w_pgT, in_sems, send_sems, recv_sems):
        my_x = lax.axis_index("x")
        my_y = lax.axis_index("y")
        nbr = (1 - my_x, my_y)

        cp_k = pltpu.make_async_copy(k_hbm, k_vm, in_sems.at[3])
        cp_v = pltpu.make_async_copy(v_hbm, v_vm, in_sems.at[4])
        cp_q = pltpu.make_async_copy(q_hbm, q_vm, in_sems.at[0])
        cp_bt = pltpu.make_async_copy(bt_hbm, bt_vm, in_sems.at[1])
        cp_ln = pltpu.make_async_copy(lens_hbm, lens_sm, in_sems.at[2])
        for c in (cp_k, cp_v, cp_q, cp_bt, cp_ln):
            c.start()

        barrier = pltpu.get_barrier_semaphore()
        pl.semaphore_signal(
            barrier, inc=1,
            device_id=nbr, device_id_type=pl.DeviceIdType.MESH,
        )
        pl.semaphore_wait(barrier, 1)

        cp_bt.wait()
        cp_ln.wait()
        p_col = my_x * PAGES_LOCAL + lax.broadcasted_iota(
            jnp.int32, (PAGES_LOCAL, 1), 0
        )
        slot_row = lax.broadcasted_iota(jnp.int32, (1, SLOTS), 1)
        for b in range(B):
            bt_row = bt_vm[b : b + 1, :]
            valid_row = slot_row < lens_sm[b]
            cmp = (bt_row == p_col) & valid_row
            w_pgT[:, b : b + 1] = jnp.sum(
                cmp.astype(jnp.float32), axis=1, keepdims=True
            )
        exp_mat = (
            lax.broadcasted_iota(jnp.int32, (PAGES_LOCAL, T), 0)
            == lax.broadcasted_iota(jnp.int32, (PAGES_LOCAL, T), 1) // BS
        ).astype(jnp.float32)
        w_tokT = lax.dot_general(
            exp_mat, w_pgT[:], (((0,), (0,)), ((), ())),
            preferred_element_type=jnp.float32,
        )
        lnwT = jnp.where(w_tokT > 0.0, jnp.log(w_tokT), NEG)
        lnw_all = jnp.concatenate([lnwT] * H, axis=1)

        cp_q.wait()
        cp_k.wait()
        k2 = k_vm[:].reshape(T, H * D)
        scale = D ** -0.5
        s_parts = []
        for h in range(H):
            q_h = q_vm[:, 0, h, :]
            s_parts.append(
                lax.dot_general(
                    k2[:, h * D : (h + 1) * D], q_h,
                    (((1,), (1,)), ((), ())),
                    preferred_element_type=jnp.float32,
                )
            )
        S = jnp.concatenate(s_parts, axis=1)
        P = jnp.exp(S * scale + lnw_all)
        cp_v.wait()
        v_aug = jnp.concatenate(
            [v_vm[:].reshape(T, H * D), jnp.ones((T, 1), jnp.float32)],
            axis=1,
        )

        rdmas = []
        for g in range(2):
            Og = lax.dot_general(
                P[:, g * (HALF * B) : (g + 1) * (HALF * B)], v_aug,
                (((0,), (0,)), ((), ())),
                preferred_element_type=jnp.float32,
            )
            for hl in range(HALF):
                h = g * HALF + hl
                lo = hl * B
                o_buf[0, h, :, :D] = Og[lo : lo + B, h * D : (h + 1) * D]
                o_buf[0, h, :, D : D + 1] = Og[lo : lo + B, H * D : H * D + 1]
            rdma = pltpu.make_async_remote_copy(
                src_ref=o_buf.at[0, pl.ds(g * HALF, HALF)],
                dst_ref=o_buf.at[1, pl.ds(g * HALF, HALF)],
                send_sem=send_sems.at[g],
                recv_sem=recv_sems.at[g],
                device_id=nbr,
                device_id_type=pl.DeviceIdType.MESH,
            )
            rdma.start()
            rdmas.append(rdma)

        for g in range(2):
            rdmas[g].wait()
            for hl in range(HALF):
                h = g * HALF + hl
                s0 = o_buf[0, h]
                s1 = o_buf[1, h]
                num = s0[:, :D] + s1[:, :D]
                den = s0[:, D : D + 1] + s1[:, D : D + 1]
                out_vm[:, 0, h, :] = num / den

        cp_out = pltpu.make_async_copy(out_vm, out_hbm, in_sems.at[0])
        cp_out.start()
        cp_out.wait()

    return pl.pallas_call(
        body,
        out_shape=jax.ShapeDtypeStruct((B, 1, H, D), jnp.float32),
        in_specs=[pl.BlockSpec(memory_space=pltpu.MemorySpace.HBM)] * 5,
        out_specs=pl.BlockSpec(memory_space=pltpu.MemorySpace.HBM),
        scratch_shapes=[
            pltpu.VMEM((B, 1, H, D), jnp.float32),
            pltpu.VMEM((PAGES_LOCAL, BS, H, D), jnp.float32),
            pltpu.VMEM((PAGES_LOCAL, BS, H, D), jnp.float32),
            pltpu.VMEM((B, SLOTS), jnp.int32),
            pltpu.SMEM((B,), jnp.int32),
            pltpu.VMEM((B, 1, H, D), jnp.float32),
            pltpu.VMEM((2, H, B, D + 1), jnp.float32),
            pltpu.VMEM((PAGES_LOCAL, B), jnp.float32),
            pltpu.SemaphoreType.DMA((5,)),
            pltpu.SemaphoreType.DMA((2,)),
            pltpu.SemaphoreType.DMA((2,)),
        ],
        compiler_params=pltpu.CompilerParams(collective_id=0),
    )(Q, K, V, bt, lens)


# device time: 16934 ns/iter; 1.0619x vs baseline; 1.0619x over previous
import jax
import jax.numpy as jnp
from jax import lax
from jax.experimental import pallas as pl
from jax.experimental.pallas import tpu as pltpu

B, H, D, BS = 8, 8, 64, 16
SLOTS = 64
PAGES_LOCAL = 64
T = PAGES_LOCAL * BS
HALF = H // 2
NEG = -1e30


def kernel(Q, K, V, bt, lens):
    def body(q_vm, k_vm, v_vm, bt_vm, lens_sm, out_ref,
             o_buf, w_pgT, send_sems, recv_sems):
        my_x = lax.axis_index("x")
        my_y = lax.axis_index("y")
        nbr = (1 - my_x, my_y)

        barrier = pltpu.get_barrier_semaphore()
        pl.semaphore_signal(
            barrier, inc=1,
            device_id=nbr, device_id_type=pl.DeviceIdType.MESH,
        )
        pl.semaphore_wait(barrier, 1)

        p_col = my_x * PAGES_LOCAL + lax.broadcasted_iota(
            jnp.int32, (PAGES_LOCAL, 1), 0
        )
        slot_row = lax.broadcasted_iota(jnp.int32, (1, SLOTS), 1)
        for b in range(B):
            bt_row = bt_vm[b : b + 1, :]
            valid_row = slot_row < lens_sm[b]
            cmp = (bt_row == p_col) & valid_row
            w_pgT[:, b : b + 1] = jnp.sum(
                cmp.astype(jnp.float32), axis=1, keepdims=True
            )
        exp_mat = (
            lax.broadcasted_iota(jnp.int32, (PAGES_LOCAL, T), 0)
            == lax.broadcasted_iota(jnp.int32, (PAGES_LOCAL, T), 1) // BS
        ).astype(jnp.float32)
        w_tokT = lax.dot_general(
            exp_mat, w_pgT[:], (((0,), (0,)), ((), ())),
            preferred_element_type=jnp.float32,
        )
        lnwT = jnp.where(w_tokT > 0.0, jnp.log(w_tokT), NEG)
        lnw_all = jnp.concatenate([lnwT] * H, axis=1)

        k2 = k_vm[:].reshape(T, H * D)
        scale = D ** -0.5
        s_parts = []
        for h in range(H):
            q_h = q_vm[:, 0, h, :]
            s_parts.append(
                lax.dot_general(
                    k2[:, h * D : (h + 1) * D], q_h,
                    (((1,), (1,)), ((), ())),
                    preferred_element_type=jnp.float32,
                )
            )
        S = jnp.concatenate(s_parts, axis=1)
        P = jnp.exp(S * scale + lnw_all)
        v_aug = jnp.concatenate(
            [v_vm[:].reshape(T, H * D), jnp.ones((T, 1), jnp.float32)],
            axis=1,
        )

        rdmas = []
        for g in range(2):
            Og = lax.dot_general(
                P[:, g * (HALF * B) : (g + 1) * (HALF * B)], v_aug,
                (((0,), (0,)), ((), ())),
                preferred_element_type=jnp.float32,
            )
            for hl in range(HALF):
                h = g * HALF + hl
                lo = hl * B
                o_buf[0, h, :, :D] = Og[lo : lo + B, h * D : (h + 1) * D]
                o_buf[0, h, :, D : D + 1] = Og[lo : lo + B, H * D : H * D + 1]
            rdma = pltpu.make_async_remote_copy(
                src_ref=o_buf.at[0, pl.ds(g * HALF, HALF)],
                dst_ref=o_buf.at[1, pl.ds(g * HALF, HALF)],
                send_sem=send_sems.at[g],
                recv_sem=recv_sems.at[g],
                device_id=nbr,
                device_id_type=pl.DeviceIdType.MESH,
            )
            rdma.start()
            rdmas.append(rdma)

        for g in range(2):
            rdmas[g].wait()
            for hl in range(HALF):
                h = g * HALF + hl
                s0 = o_buf[0, h]
                s1 = o_buf[1, h]
                num = s0[:, :D] + s1[:, :D]
                den = s0[:, D : D + 1] + s1[:, D : D + 1]
                out_ref[:, 0, h, :] = num / den

    return pl.pallas_call(
        body,
        out_shape=jax.ShapeDtypeStruct((B, 1, H, D), jnp.float32),
        in_specs=[pl.BlockSpec(memory_space=pltpu.VMEM)] * 4
        + [pl.BlockSpec(memory_space=pltpu.SMEM)],
        out_specs=pl.BlockSpec(memory_space=pltpu.VMEM),
        scratch_shapes=[
            pltpu.VMEM((2, H, B, D + 1), jnp.float32),
            pltpu.VMEM((PAGES_LOCAL, B), jnp.float32),
            pltpu.SemaphoreType.DMA((2,)),
            pltpu.SemaphoreType.DMA((2,)),
        ],
        compiler_params=pltpu.CompilerParams(collective_id=0),
    )(Q, K, V, bt, lens)


# device time: 14492 ns/iter; 1.2408x vs baseline; 1.1685x over previous
import jax
import jax.numpy as jnp
from jax import lax
from jax.experimental import pallas as pl
from jax.experimental.pallas import tpu as pltpu

B, H, D, BS = 8, 8, 64, 16
SLOTS = 64
PAGES_LOCAL = 64
T = PAGES_LOCAL * BS
HALF = H // 2
NEG = -1e30


def kernel(Q, K, V, bt, lens):
    def body(q2_vm, k2_vm, v2_vm, bt_vm, lens_sm, out_ref,
             o_buf, w_pgT, send_sems, recv_sems):
        my_x = lax.axis_index("x")
        my_y = lax.axis_index("y")
        nbr = (1 - my_x, my_y)

        barrier = pltpu.get_barrier_semaphore()
        pl.semaphore_signal(
            barrier, inc=1,
            device_id=nbr, device_id_type=pl.DeviceIdType.MESH,
        )
        pl.semaphore_wait(barrier, 1)

        p_col = my_x * PAGES_LOCAL + lax.broadcasted_iota(
            jnp.int32, (PAGES_LOCAL, 1), 0
        )
        slot_row = lax.broadcasted_iota(jnp.int32, (1, SLOTS), 1)
        for b in range(B):
            bt_row = bt_vm[b : b + 1, :]
            valid_row = slot_row < lens_sm[b]
            cmp = (bt_row == p_col) & valid_row
            w_pgT[:, b : b + 1] = jnp.sum(
                cmp.astype(jnp.float32), axis=1, keepdims=True
            )
        exp_mat = (
            lax.broadcasted_iota(jnp.int32, (PAGES_LOCAL, T), 0)
            == lax.broadcasted_iota(jnp.int32, (PAGES_LOCAL, T), 1) // BS
        ).astype(jnp.float32)
        w_tokT = lax.dot_general(
            exp_mat, w_pgT[:], (((0,), (0,)), ((), ())),
            preferred_element_type=jnp.float32,
        )
        lnwT = jnp.where(w_tokT > 0.0, jnp.log(w_tokT), NEG)
        lnw_all = jnp.concatenate([lnwT] * H, axis=1)

        scale = D ** -0.5
        s_parts = []
        for h in range(H):
            s_parts.append(
                lax.dot_general(
                    k2_vm[:, h * D : (h + 1) * D],
                    q2_vm[:, h * D : (h + 1) * D],
                    (((1,), (1,)), ((), ())),
                    preferred_element_type=jnp.float32,
                )
            )
        S = jnp.concatenate(s_parts, axis=1)
        P = jnp.exp(S * scale + lnw_all)
        v_aug = jnp.concatenate(
            [v2_vm[:], jnp.ones((T, 1), jnp.float32)], axis=1
        )

        rdmas = []
        for g in range(2):
            Og = lax.dot_general(
                P[:, g * (HALF * B) : (g + 1) * (HALF * B)], v_aug,
                (((0,), (0,)), ((), ())),
                preferred_element_type=jnp.float32,
            )
            for hl in range(HALF):
                h = g * HALF + hl
                lo = hl * B
                o_buf[0, h, :, :D] = Og[lo : lo + B, h * D : (h + 1) * D]
                o_buf[0, h, :, D : D + 1] = Og[lo : lo + B, H * D : H * D + 1]
            rdma = pltpu.make_async_remote_copy(
                src_ref=o_buf.at[0, pl.ds(g * HALF, HALF)],
                dst_ref=o_buf.at[1, pl.ds(g * HALF, HALF)],
                send_sem=send_sems.at[g],
                recv_sem=recv_sems.at[g],
                device_id=nbr,
                device_id_type=pl.DeviceIdType.MESH,
            )
            rdma.start()
            rdmas.append(rdma)

        for g in range(2):
            rdmas[g].wait()
            for hl in range(HALF):
                h = g * HALF + hl
                s0 = o_buf[0, h]
                s1 = o_buf[1, h]
                num = s0[:, :D] + s1[:, :D]
                den = s0[:, D : D + 1] + s1[:, D : D + 1]
                out_ref[:, 0, h, :] = num / den

    return pl.pallas_call(
        body,
        out_shape=jax.ShapeDtypeStruct((B, 1, H, D), jnp.float32),
        in_specs=[pl.BlockSpec(memory_space=pltpu.VMEM)] * 4
        + [pl.BlockSpec(memory_space=pltpu.SMEM)],
        out_specs=pl.BlockSpec(memory_space=pltpu.VMEM),
        scratch_shapes=[
            pltpu.VMEM((2, H, B, D + 1), jnp.float32),
            pltpu.VMEM((PAGES_LOCAL, B), jnp.float32),
            pltpu.SemaphoreType.DMA((2,)),
            pltpu.SemaphoreType.DMA((2,)),
        ],
        compiler_params=pltpu.CompilerParams(collective_id=0),
    )(
        Q.reshape(B, H * D),
        K.reshape(T, H * D),
        V.reshape(T, H * D),
        bt,
        lens,
    )


# device time: 13828 ns/iter; 1.3004x vs baseline; 1.0480x over previous
import jax
import jax.numpy as jnp
from jax import lax
from jax.experimental import pallas as pl
from jax.experimental.pallas import tpu as pltpu

B, H, D, BS = 8, 8, 64, 16
SLOTS = 64
PAGES_LOCAL = 64
T = PAGES_LOCAL * BS
HALF = H // 2
NEG = -1e30


def kernel(Q, K, V, bt, lens):
    def body(q_vm, k2_vm, v2_vm, bt_vm, lens_sm, out_ref,
             o_buf, w_pgT, send_sems, recv_sems):
        my_x = lax.axis_index("x")
        my_y = lax.axis_index("y")
        nbr = (1 - my_x, my_y)

        barrier = pltpu.get_barrier_semaphore()
        pl.semaphore_signal(
            barrier, inc=1,
            device_id=nbr, device_id_type=pl.DeviceIdType.MESH,
        )
        pl.semaphore_wait(barrier, 1)

        p_col = my_x * PAGES_LOCAL + lax.broadcasted_iota(
            jnp.int32, (PAGES_LOCAL, 1), 0
        )
        slot_row = lax.broadcasted_iota(jnp.int32, (1, SLOTS), 1)
        for b in range(B):
            bt_row = bt_vm[b : b + 1, :]
            valid_row = slot_row < lens_sm[b]
            cmp = (bt_row == p_col) & valid_row
            w_pgT[:, b : b + 1] = jnp.sum(
                cmp.astype(jnp.float32), axis=1, keepdims=True
            )
        exp_mat = (
            lax.broadcasted_iota(jnp.int32, (PAGES_LOCAL, T), 0)
            == lax.broadcasted_iota(jnp.int32, (PAGES_LOCAL, T), 1) // BS
        ).astype(jnp.float32)
        w_tokT = lax.dot_general(
            exp_mat, w_pgT[:], (((0,), (0,)), ((), ())),
            preferred_element_type=jnp.float32,
        )
        lnwT = jnp.where(w_tokT > 0.0, jnp.log(w_tokT), NEG)
        lnw_all = jnp.concatenate([lnwT] * H, axis=1)

        scale = D ** -0.5
        s_parts = []
        for h in range(H):
            s_parts.append(
                lax.dot_general(
                    k2_vm[:, h * D : (h + 1) * D],
                    q_vm[:, 0, h, :].astype(jnp.bfloat16),
                    (((1,), (1,)), ((), ())),
                    preferred_element_type=jnp.float32,
                )
            )
        S = jnp.concatenate(s_parts, axis=1)
        P = jnp.exp(S * scale + lnw_all)
        v_aug = jnp.concatenate(
            [v2_vm[:], jnp.ones((T, 1), jnp.bfloat16)], axis=1
        )

        rdmas = []
        for g in range(2):
            Og = lax.dot_general(
                P[:, g * (HALF * B) : (g + 1) * (HALF * B)].astype(jnp.bfloat16),
                v_aug,
                (((0,), (0,)), ((), ())),
                preferred_element_type=jnp.float32,
            )
            for hl in range(HALF):
                h = g * HALF + hl
                lo = hl * B
                o_buf[0, h, :, :D] = Og[lo : lo + B, h * D : (h + 1) * D]
                o_buf[0, h, :, D : D + 1] = Og[lo : lo + B, H * D : H * D + 1]
            rdma = pltpu.make_async_remote_copy(
                src_ref=o_buf.at[0, pl.ds(g * HALF, HALF)],
                dst_ref=o_buf.at[1, pl.ds(g * HALF, HALF)],
                send_sem=send_sems.at[g],
                recv_sem=recv_sems.at[g],
                device_id=nbr,
                device_id_type=pl.DeviceIdType.MESH,
            )
            rdma.start()
            rdmas.append(rdma)

        for g in range(2):
            rdmas[g].wait()
            for hl in range(HALF):
                h = g * HALF + hl
                s0 = o_buf[0, h]
                s1 = o_buf[1, h]
                num = s0[:, :D] + s1[:, :D]
                den = s0[:, D : D + 1] + s1[:, D : D + 1]
                out_ref[:, 0, h, :] = num / den

    return pl.pallas_call(
        body,
        out_shape=jax.ShapeDtypeStruct((B, 1, H, D), jnp.float32),
        in_specs=[pl.BlockSpec(memory_space=pltpu.VMEM)] * 4
        + [pl.BlockSpec(memory_space=pltpu.SMEM)],
        out_specs=pl.BlockSpec(memory_space=pltpu.VMEM),
        scratch_shapes=[
            pltpu.VMEM((2, H, B, D + 1), jnp.float32),
            pltpu.VMEM((PAGES_LOCAL, B), jnp.float32),
            pltpu.SemaphoreType.DMA((2,)),
            pltpu.SemaphoreType.DMA((2,)),
        ],
        compiler_params=pltpu.CompilerParams(collective_id=0),
    )(
        Q,
        K.reshape(T, H * D).astype(jnp.bfloat16),
        V.reshape(T, H * D).astype(jnp.bfloat16),
        bt,
        lens,
    )
